# Initial kernel scaffold; baseline (speedup 1.0000x reference)
import jax
import jax.numpy as jnp
from jax.experimental import pallas as pl


def kernel(image_emb):
    raise NotImplementedError("write your pallas kernel here")


# speedup vs baseline: 8.4089x; 8.4089x over previous
import jax
import jax.numpy as jnp
from jax.experimental import pallas as pl
from jax.experimental.pallas import tpu as pltpu

_N_CLUSTERS = 9
_N_ITER = 10
_COMPACTNESS = 3.0
_H = 24
_N = _H * _H
_D_EMB = 768
_D = _D_EMB + 2
_G = 8

_NT = (((1,), (1,)), ((), ()))
_NN = (((1,), (0,)), ((), ()))


def _mm(a, b, dims):
    return jax.lax.dot_general(a, b, dims, preferred_element_type=jnp.float32)


def _ssn_kernel(emb_ref, out_ref, gram_ref):
    col = jax.lax.broadcasted_iota(jnp.int32, (_N, 2), 1)
    n = jax.lax.broadcasted_iota(jnp.int32, (_N, 2), 0)
    x = (n % _H).astype(jnp.float32) * _COMPACTNESS
    y = (n // _H).astype(jnp.float32) * _COMPACTNESS
    coords = jnp.where(col == 0, x, y)
    cgram = _mm(coords, coords, _NT)

    for g in range(_G):
        gram_ref[g] = _mm(emb_ref[g], emb_ref[g], _NT) + cgram

    kk = jax.lax.broadcasted_iota(jnp.int32, (_N_CLUSTERS, _N), 0)
    nn = jax.lax.broadcasted_iota(jnp.int32, (_N_CLUSTERS, _N), 1)
    blk = (nn // _H // 8) * 3 + (nn % _H) // 8
    at0 = (blk == kk).astype(jnp.float32)

    qt = [at0] * _G
    rr = [jnp.full((_N_CLUSTERS, 1), 1.0 / (64.0 + 1e-8), jnp.float32)] * _G
    for it in range(_N_ITER):
        w = [_mm(qt[g], gram_ref[g], _NN) for g in range(_G)]
        for g in range(_G):
            cwq = jnp.sum(w[g] * qt[g], axis=1, keepdims=True)
            s = rr[g] * (w[g] + w[g] - rr[g] * cwq)
            e = jnp.exp(s - jnp.max(s, axis=0, keepdims=True))
            qt[g] = e * (1.0 / jnp.sum(e, axis=0, keepdims=True))
            if it + 1 < _N_ITER:
                denom = jnp.sum(qt[g], axis=1, keepdims=True)
                rr[g] = 1.0 / (denom + 1e-8)

    for g in range(_G):
        out_ref[g] = qt[g].T


@jax.jit
def kernel(image_emb):
    b = image_emb.shape[0]
    return pl.pallas_call(
        _ssn_kernel,
        grid=(b // _G,),
        in_specs=[pl.BlockSpec((_G, _N, _D_EMB), lambda i: (i, 0, 0))],
        out_specs=pl.BlockSpec((_G, _N, _N_CLUSTERS), lambda i: (i, 0, 0)),
        out_shape=jax.ShapeDtypeStruct((b, _N, _N_CLUSTERS), jnp.float32),
        scratch_shapes=[
            pltpu.VMEM((_G, _N, _N), jnp.float32),
        ],
        compiler_params=pltpu.CompilerParams(
            dimension_semantics=("parallel",)),
    )(image_emb)
